# baseline (speedup 1.0000x reference)
import functools

import jax
import jax.numpy as jnp
from jax import lax
from jax.experimental import pallas as pl
from jax.experimental.pallas import tpu as pltpu
from jax.experimental.pallas import tpu_sc as plsc

N = 10000
E = 320000
NC = 2
NS = 16
NW = NC * NS
EPW = E // NW
CHUNK = 80
NCHUNK = EPW // CHUNK
NP = 10112
RPT = NP // NS

NBUF = 4
GE = NBUF * CHUNK
NGF = NCHUNK // NBUF


def _make_scatter(d):
    mesh = plsc.VectorSubcoreMesh(
        core_axis_name="c", subcore_axis_name="s",
        num_cores=NC, num_subcores=NS)

    @functools.partial(
        pl.kernel,
        out_type=jax.ShapeDtypeStruct((NC, NP, d), jnp.float32),
        mesh=mesh,
        scratch_types=[
            [pltpu.VMEM((GE,), jnp.int32)] * 2,
            [pltpu.VMEM((GE,), jnp.int32)] * 2,
            [pltpu.VMEM((CHUNK, d), jnp.float32)] * NBUF,
            [pltpu.SemaphoreType.DMA] * NBUF,
            [pltpu.SemaphoreType.DMA] * NBUF,
            [pltpu.SemaphoreType.DMA] * 2,
            pltpu.SemaphoreType.DMA,
            pltpu.VMEM_SHARED((NP, d), jnp.float32),
        ],
    )
    def scatter(m_hbm, src_hbm, dst_hbm, zeros_hbm, out_hbm,
                srcg, dstg, rows, gs, ss, isem, zsem, acc):
        c = lax.axis_index("c")
        s = lax.axis_index("s")
        wid = s * NC + c
        ebase = wid * EPW
        zc = pltpu.async_copy(zeros_hbm.at[pl.ds(s * RPT, RPT), :],
                              acc.at[pl.ds(s * RPT, RPT), :], zsem)
        pltpu.async_copy(src_hbm.at[pl.ds(ebase, GE)], srcg[0], isem[0])
        pltpu.async_copy(dst_hbm.at[pl.ds(ebase, GE)], dstg[0], isem[0])
        pltpu.async_copy(src_hbm.at[pl.ds(ebase + GE, GE)], srcg[1], isem[1])
        pltpu.async_copy(dst_hbm.at[pl.ds(ebase + GE, GE)], dstg[1], isem[1])

        def wait_idx(p):
            pltpu.make_async_copy(
                src_hbm.at[pl.ds(0, GE)], srcg[p], isem[p]).wait()
            pltpu.make_async_copy(
                dst_hbm.at[pl.ds(0, GE)], dstg[p], isem[p]).wait()

        def start_gather(p, k, b):
            pltpu.async_copy(
                m_hbm.at[srcg[p].at[pl.ds(k * CHUNK, CHUNK)]], rows[b],
                gs[b])

        def wait_gather(b):
            pltpu.make_async_copy(
                m_hbm.at[srcg[0].at[pl.ds(0, CHUNK)]], rows[b], gs[b]).wait()

        def start_scatter(p, k, b):
            pltpu.async_copy(
                rows[b], acc.at[dstg[p].at[pl.ds(k * CHUNK, CHUNK)]], ss[b],
                add=True)

        def wait_scatter(b):
            pltpu.make_async_copy(
                rows[b], acc.at[dstg[0].at[pl.ds(0, CHUNK)]], ss[b]).wait()

        wait_idx(0)
        for b in range(NBUF):
            start_gather(0, b, b)
        zc.wait()
        plsc.subcore_barrier()

        last_off = ebase + EPW - GE

        def group_body(g, p):
            for b in range(NBUF):
                wait_gather(b)
                start_scatter(p, b, b)
            wait_idx(p ^ 1)
            for b in range(NBUF):
                wait_scatter(b)
                start_gather(p ^ 1, b, b)
            off = jnp.minimum(ebase + (g + 2) * GE, last_off)
            pltpu.async_copy(src_hbm.at[pl.ds(off, GE)], srcg[p], isem[p])
            pltpu.async_copy(dst_hbm.at[pl.ds(off, GE)], dstg[p], isem[p])

        def pair(gp, carry):
            group_body(2 * gp, 0)
            group_body(2 * gp + 1, 1)
            return carry

        lax.fori_loop(0, NGF // 2, pair, 0)

        for b in range(NBUF):
            wait_gather(b)
            start_scatter(0, b, b)
        wait_idx(1)
        wait_scatter(0)
        start_gather(1, NBUF - 1, 0)
        for b in range(1, NBUF):
            wait_scatter(b)
        wait_gather(0)
        start_scatter(1, NBUF - 1, 0)
        wait_scatter(0)

        plsc.subcore_barrier()
        pltpu.sync_copy(acc.at[pl.ds(s * RPT, RPT), :],
                        out_hbm.at[c, pl.ds(s * RPT, RPT), :])

    return scatter


_scatter128 = _make_scatter(128)


def _bn_body(p_ref, w1_ref, b1_ref, g_ref, be_ref, o_ref):
    h = jnp.dot(p_ref[0, :N] + p_ref[1, :N], w1_ref[...],
                preferred_element_type=jnp.float32) + b1_ref[...]
    mean = jnp.mean(h, axis=0, keepdims=True)
    var = jnp.mean((h - mean) ** 2, axis=0, keepdims=True)
    h = (h - mean) * lax.rsqrt(var + 1e-5) * g_ref[...] + be_ref[...]
    o_ref[...] = jnp.maximum(h, 0.0)


def _mm_norm_body(p_ref, w2_ref, b2_ref, o_ref):
    h = jnp.dot(p_ref[0, :N] + p_ref[1, :N], w2_ref[...],
                preferred_element_type=jnp.float32) + b2_ref[...]
    nrm = jnp.sqrt(jnp.sum(h * h, axis=1, keepdims=True))
    o_ref[...] = h / jnp.maximum(nrm, 1e-12)


def kernel(x, edge_index, W1, b1, gamma, beta, W2, b2):
    src = edge_index[0]
    dst = edge_index[1]
    z128 = jnp.zeros((NP, 128), jnp.float32)

    p1 = _scatter128(x, src, dst, z128)

    h = pl.pallas_call(
        _bn_body,
        out_shape=jax.ShapeDtypeStruct((N, 128), jnp.float32),
    )(p1, W1, b1.reshape(1, -1), gamma.reshape(1, -1), beta.reshape(1, -1))

    p2 = _scatter128(h, src, dst, z128)

    return pl.pallas_call(
        _mm_norm_body,
        out_shape=jax.ShapeDtypeStruct((N, 64), jnp.float32),
    )(p2, W2, b2.reshape(1, -1))


# speedup vs baseline: 1.0243x; 1.0243x over previous
import functools

import jax
import jax.numpy as jnp
from jax import lax
from jax.experimental import pallas as pl
from jax.experimental.pallas import tpu as pltpu
from jax.experimental.pallas import tpu_sc as plsc

N = 10000
E = 320000
NC = 2
NS = 16
NW = NC * NS
EPW = E // NW
CHUNK = 80
NCHUNK = EPW // CHUNK
NP = 10112
RPT = NP // NS

NBUF = 4
GE = NBUF * CHUNK
NGF = NCHUNK // NBUF
ZR = 48


def _make_scatter(d):
    mesh = plsc.VectorSubcoreMesh(
        core_axis_name="c", subcore_axis_name="s",
        num_cores=NC, num_subcores=NS)

    @functools.partial(
        pl.kernel,
        out_type=jax.ShapeDtypeStruct((NC, NP, d), jnp.float32),
        mesh=mesh,
        scratch_types=[
            [pltpu.VMEM((GE,), jnp.int32)] * 2,
            [pltpu.VMEM((GE,), jnp.int32)] * 2,
            [pltpu.VMEM((CHUNK, d), jnp.float32)] * NBUF,
            [pltpu.SemaphoreType.DMA] * NBUF,
            [pltpu.SemaphoreType.DMA] * NBUF,
            [pltpu.SemaphoreType.DMA] * 2,
            pltpu.SemaphoreType.DMA,
            pltpu.VMEM((ZR, d), jnp.float32),
            pltpu.VMEM_SHARED((NP, d), jnp.float32),
        ],
    )
    def scatter(m_hbm, src_hbm, dst_hbm, out_hbm,
                srcg, dstg, rows, gs, ss, isem, zsem, zbuf, acc):
        c = lax.axis_index("c")
        s = lax.axis_index("s")
        wid = s * NC + c
        ebase = wid * EPW
        pltpu.async_copy(src_hbm.at[pl.ds(ebase, GE)], srcg[0], isem[0])
        pltpu.async_copy(dst_hbm.at[pl.ds(ebase, GE)], dstg[0], isem[0])
        pltpu.async_copy(src_hbm.at[pl.ds(ebase + GE, GE)], srcg[1], isem[1])
        pltpu.async_copy(dst_hbm.at[pl.ds(ebase + GE, GE)], dstg[1], isem[1])

        zv = jnp.zeros((16,), jnp.float32)
        for r in range(ZR):
            for q in range(d // 16):
                zbuf[r, pl.ds(q * 16, 16)] = zv
        for i in range(RPT // ZR):
            pltpu.async_copy(
                zbuf, acc.at[pl.ds(s * RPT + i * ZR, ZR), :], zsem)
        pltpu.async_copy(
            zbuf.at[pl.ds(0, RPT % ZR), :],
            acc.at[pl.ds(s * RPT + (RPT // ZR) * ZR, RPT % ZR), :], zsem)

        def wait_idx(p):
            pltpu.make_async_copy(
                src_hbm.at[pl.ds(0, GE)], srcg[p], isem[p]).wait()
            pltpu.make_async_copy(
                dst_hbm.at[pl.ds(0, GE)], dstg[p], isem[p]).wait()

        def start_gather(p, k, b):
            pltpu.async_copy(
                m_hbm.at[srcg[p].at[pl.ds(k * CHUNK, CHUNK)]], rows[b],
                gs[b])

        def wait_gather(b):
            pltpu.make_async_copy(
                m_hbm.at[srcg[0].at[pl.ds(0, CHUNK)]], rows[b], gs[b]).wait()

        def start_scatter(p, k, b):
            pltpu.async_copy(
                rows[b], acc.at[dstg[p].at[pl.ds(k * CHUNK, CHUNK)]], ss[b],
                add=True)

        def wait_scatter(b):
            pltpu.make_async_copy(
                rows[b], acc.at[dstg[0].at[pl.ds(0, CHUNK)]], ss[b]).wait()

        wait_idx(0)
        for b in range(NBUF):
            start_gather(0, b, b)
        for i in range(RPT // ZR):
            pltpu.make_async_copy(
                zbuf, acc.at[pl.ds(s * RPT, ZR), :], zsem).wait()
        pltpu.make_async_copy(
            zbuf.at[pl.ds(0, RPT % ZR), :],
            acc.at[pl.ds(s * RPT, RPT % ZR), :], zsem).wait()
        plsc.subcore_barrier()

        last_off = ebase + EPW - GE

        def group_body(g, p):
            for b in range(NBUF):
                wait_gather(b)
                start_scatter(p, b, b)
            wait_idx(p ^ 1)
            for b in range(NBUF):
                wait_scatter(b)
                start_gather(p ^ 1, b, b)
            off = jnp.minimum(ebase + (g + 2) * GE, last_off)
            pltpu.async_copy(src_hbm.at[pl.ds(off, GE)], srcg[p], isem[p])
            pltpu.async_copy(dst_hbm.at[pl.ds(off, GE)], dstg[p], isem[p])

        def pair(gp, carry):
            group_body(2 * gp, 0)
            group_body(2 * gp + 1, 1)
            return carry

        lax.fori_loop(0, NGF // 2, pair, 0)

        for b in range(NBUF):
            wait_gather(b)
            start_scatter(0, b, b)
        wait_idx(1)
        wait_scatter(0)
        start_gather(1, NBUF - 1, 0)
        for b in range(1, NBUF):
            wait_scatter(b)
        wait_gather(0)
        start_scatter(1, NBUF - 1, 0)
        wait_scatter(0)

        plsc.subcore_barrier()
        pltpu.sync_copy(acc.at[pl.ds(s * RPT, RPT), :],
                        out_hbm.at[c, pl.ds(s * RPT, RPT), :])

    return scatter


_scatter128 = _make_scatter(128)


def _bn_body(p_ref, w1_ref, b1_ref, g_ref, be_ref, o_ref):
    h = jnp.dot(p_ref[0, :N] + p_ref[1, :N], w1_ref[...],
                preferred_element_type=jnp.float32) + b1_ref[...]
    mean = jnp.mean(h, axis=0, keepdims=True)
    var = jnp.mean((h - mean) ** 2, axis=0, keepdims=True)
    h = (h - mean) * lax.rsqrt(var + 1e-5) * g_ref[...] + be_ref[...]
    o_ref[...] = jnp.maximum(h, 0.0)


def _mm_norm_body(p_ref, w2_ref, b2_ref, o_ref):
    h = jnp.dot(p_ref[0, :N] + p_ref[1, :N], w2_ref[...],
                preferred_element_type=jnp.float32) + b2_ref[...]
    nrm = jnp.sqrt(jnp.sum(h * h, axis=1, keepdims=True))
    o_ref[...] = h / jnp.maximum(nrm, 1e-12)


def kernel(x, edge_index, W1, b1, gamma, beta, W2, b2):
    src = edge_index[0]
    dst = edge_index[1]

    p1 = _scatter128(x, src, dst)

    h = pl.pallas_call(
        _bn_body,
        out_shape=jax.ShapeDtypeStruct((N, 128), jnp.float32),
    )(p1, W1, b1.reshape(1, -1), gamma.reshape(1, -1), beta.reshape(1, -1))

    p2 = _scatter128(h, src, dst)

    return pl.pallas_call(
        _mm_norm_body,
        out_shape=jax.ShapeDtypeStruct((N, 64), jnp.float32),
    )(p2, W2, b2.reshape(1, -1))
